# baseline (speedup 1.0000x reference)
import functools

import jax
import jax.numpy as jnp
from jax import lax
from jax.experimental import pallas as pl
from jax.experimental.pallas import tpu as pltpu
from jax.experimental.pallas import tpu_sc as plsc

_GENE_VOCAB = 60697
_VALUE_VOCAB = 51
_D = 512
_B, _L = 64, 1200
_N = _B * _L
_NC, _NS = 2, 16
_NW = _NC * _NS
_PER_W = _N // _NW
_C = 32
_NCHUNK = _PER_W // _C
_NB = 3
_NR = _NCHUNK // _NB

_mesh = plsc.VectorSubcoreMesh(core_axis_name="c", subcore_axis_name="s")


@functools.partial(
    pl.kernel,
    mesh=_mesh,
    out_type=jax.ShapeDtypeStruct((_N, _D), jnp.float32),
    compiler_params=pltpu.CompilerParams(needs_layout_passes=False),
    scratch_types=[
        pltpu.VMEM((_PER_W,), jnp.int32),
        pltpu.VMEM((_PER_W,), jnp.int32),
        pltpu.VMEM((_C, _D), jnp.float32),
        pltpu.VMEM((_C, _D), jnp.float32),
        pltpu.VMEM((_C, _D), jnp.float32),
        pltpu.VMEM((_VALUE_VOCAB * _D,), jnp.float32),
        pltpu.SemaphoreType.DMA,
        pltpu.SemaphoreType.DMA,
        pltpu.SemaphoreType.DMA,
        pltpu.SemaphoreType.DMA,
        pltpu.SemaphoreType.DMA,
        pltpu.SemaphoreType.DMA,
    ],
)
def _sc_embed(ids_hbm, vals_hbm, gene_hbm, vtab_hbm, out_hbm,
              gidx, vidx, g0, g1, g2, vtab_l,
              sg0, sg1, sg2, so0, so1, so2):
    wid = lax.axis_index("s") * _NC + lax.axis_index("c")
    base = wid * _PER_W

    pltpu.sync_copy(vtab_hbm, vtab_l)
    pltpu.sync_copy(ids_hbm.at[pl.ds(base, _PER_W)], gidx)
    pltpu.sync_copy(vals_hbm.at[pl.ds(base, _PER_W)], vidx)

    gbufs = (g0, g1, g2)
    gsems = (sg0, sg1, sg2)
    osems = (so0, so1, so2)
    lanes = lax.iota(jnp.int32, 16)

    def issue_gather(ci, i):
        isl = pl.ds(pl.multiple_of(ci * _C, 8), _C)
        pltpu.async_copy(gene_hbm.at[gidx.at[isl]], gbufs[i], gsems[i])

    def wait_gather(ci, i):
        isl = pl.ds(pl.multiple_of(ci * _C, 8), _C)
        pltpu.make_async_copy(gene_hbm.at[gidx.at[isl]], gbufs[i], gsems[i]).wait()

    def out_slice(ci):
        return out_hbm.at[pl.ds(pl.multiple_of(base + ci * _C, 8), _C)]

    def add_values(ci, gbuf):
        cbase = ci * _C

        def row_body(t, carry):
            for u in range(2):
                r = 2 * t + u
                splat = jnp.full((16,), cbase + r, jnp.int32)
                sv = plsc.load_gather(vidx, [splat])
                vbase = sv * _D + lanes
                for j in range(_D // 16):
                    sl = pl.ds(j * 16, 16)
                    vrow = plsc.load_gather(vtab_l, [vbase + j * 16])
                    gbuf[r, sl] = gbuf[r, sl] + vrow
            return carry

        lax.fori_loop(0, _C // 2, row_body, 0)

    issue_gather(0, 0)
    issue_gather(1, 1)

    def round_body(k, carry):
        for i in range(_NB):
            x = _NB * k + i

            wait_gather(x, i)
            add_values(x, gbufs[i])
            pltpu.async_copy(gbufs[i], out_slice(x), osems[i])

            prev = (i + _NB - 1) % _NB
            if i == 0:
                @pl.when(k > 0)
                def _():
                    pltpu.make_async_copy(gbufs[prev], out_slice(x - 1),
                                          osems[prev]).wait()

                issue_gather(x + 2, prev)
            else:
                @pl.when(x + 2 < _NCHUNK)
                def _():
                    pltpu.make_async_copy(gbufs[prev], out_slice(x - 1),
                                          osems[prev]).wait()
                    issue_gather(x + 2, prev)
        return carry

    lax.fori_loop(0, _NR, round_body, 0)
    pltpu.make_async_copy(g0, out_slice(_NCHUNK - 3), so0).wait()
    pltpu.make_async_copy(g1, out_slice(_NCHUNK - 2), so1).wait()
    pltpu.make_async_copy(g2, out_slice(_NCHUNK - 1), so2).wait()


def kernel(input_ids, values, gene_table, value_table):
    ids = input_ids.reshape(-1).astype(jnp.int32)
    vals = values.reshape(-1).astype(jnp.int32)
    out = _sc_embed(ids, vals, gene_table, value_table.reshape(-1))
    return out.reshape(_B, _L, _D)


# speedup vs baseline: 1.3510x; 1.3510x over previous
import functools

import jax
import jax.numpy as jnp
import numpy as np
from jax import lax
from jax.experimental import pallas as pl
from jax.experimental.pallas import tpu as pltpu
from jax.experimental.pallas import tpu_sc as plsc

_GENE_VOCAB = 60697
_VALUE_VOCAB = 51
_VIN = 56
_VPAD = 112
_D = 512
_B, _L = 64, 1200
_N = _B * _L
_NC, _NS = 2, 16
_NW = _NC * _NS
_PER_W = _N // _NW
_C = 48
_NCHUNK = _PER_W // _C
_NK = _NCHUNK // 2

_PERM = np.concatenate([
    32 * b + np.stack([np.arange(16), np.arange(16) + 16], axis=1).T.reshape(-1)
    for b in range(_D // 32)
])

_mesh = plsc.VectorSubcoreMesh(core_axis_name="c", subcore_axis_name="s")


@functools.partial(
    pl.kernel,
    mesh=_mesh,
    out_type=jax.ShapeDtypeStruct((_N, _D), jnp.float32),
    scratch_types=[
        pltpu.VMEM((_PER_W,), jnp.int32),
        pltpu.VMEM((_PER_W,), jnp.int32),
        pltpu.VMEM((_C, _D), jnp.float32),
        pltpu.VMEM((_C, _D // 2), jnp.int32),
        pltpu.VMEM((_C, _D), jnp.float32),
        pltpu.VMEM((_C, _D // 2), jnp.int32),
        pltpu.HBM((_NW * _VPAD, _D // 2), jnp.int32),
        pltpu.SemaphoreType.DMA,
        pltpu.SemaphoreType.DMA,
        pltpu.SemaphoreType.DMA,
        pltpu.SemaphoreType.DMA,
        pltpu.SemaphoreType.DMA,
        pltpu.SemaphoreType.DMA,
    ],
)
def _sc_embed(ids_hbm, vals_hbm, gene_hbm, vtab_hbm, out_hbm,
              gidx, vidx, g0, v0, g1, v1, vrep,
              sg0, sv0, sg1, sv1, so0, so1):
    wid = lax.axis_index("s") * _NC + lax.axis_index("c")
    base = wid * _PER_W

    pltpu.sync_copy(vtab_hbm.at[pl.ds(0, _C)], v0)
    pltpu.sync_copy(v0, vrep.at[pl.ds(wid * _VPAD, _C)])
    pltpu.sync_copy(vtab_hbm.at[pl.ds(_C, _VIN - _C)], v0.at[pl.ds(0, _VIN - _C)])
    pltpu.sync_copy(v0.at[pl.ds(0, _VIN - _C)], vrep.at[pl.ds(wid * _VPAD + _C, _VIN - _C)])

    pltpu.sync_copy(ids_hbm.at[pl.ds(base, _PER_W)], gidx)
    pltpu.sync_copy(vals_hbm.at[pl.ds(base, _PER_W)], vidx)

    vbase = wid * _VPAD

    def rebase(i, carry):
        sl = pl.ds(pl.multiple_of(i * 16, 16), 16)
        vidx[sl] = vidx[sl] + vbase
        return carry

    lax.fori_loop(0, _PER_W // 16, rebase, 0)

    def issue_gathers(ci, gbuf, vbuf, sg, sv):
        isl = pl.ds(pl.multiple_of(ci * _C, _C), _C)
        pltpu.async_copy(gene_hbm.at[gidx.at[isl]], gbuf, sg)
        pltpu.async_copy(vrep.at[vidx.at[isl]], vbuf, sv)

    def wait_gathers(ci, gbuf, vbuf, sg, sv):
        isl = pl.ds(pl.multiple_of(ci * _C, _C), _C)
        pltpu.make_async_copy(gene_hbm.at[gidx.at[isl]], gbuf, sg).wait()
        pltpu.make_async_copy(vrep.at[vidx.at[isl]], vbuf, sv).wait()

    def out_slice(ci):
        return out_hbm.at[pl.ds(pl.multiple_of(base + ci * _C, _C), _C)]

    def add_rows(gbuf, vbuf):
        def body(r, carry):
            for q in range(_D // 32):
                w = vbuf[r, pl.ds(q * 16, 16)]
                a = lax.bitcast_convert_type(w << 16, jnp.float32)
                b = lax.bitcast_convert_type(w & jnp.int32(-65536), jnp.float32)
                c0 = q * 32
                sla = pl.ds(c0, 16)
                slb = pl.ds(c0 + 16, 16)
                gbuf[r, sla] = gbuf[r, sla] + a
                gbuf[r, slb] = gbuf[r, slb] + b
            return carry
        lax.fori_loop(0, _C, body, 0)

    issue_gathers(0, g0, v0, sg0, sv0)

    def round_body(k, carry):
        a = 2 * k
        b = a + 1

        @pl.when(k > 0)
        def _():
            pltpu.make_async_copy(g1, out_slice(b - 2), so1).wait()

        issue_gathers(b, g1, v1, sg1, sv1)

        wait_gathers(a, g0, v0, sg0, sv0)
        add_rows(g0, v0)
        pltpu.async_copy(g0, out_slice(a), so0)

        @pl.when(k < _NK - 1)
        def _():
            pltpu.make_async_copy(g0, out_slice(a), so0).wait()
            issue_gathers(a + 2, g0, v0, sg0, sv0)

        wait_gathers(b, g1, v1, sg1, sv1)
        add_rows(g1, v1)
        pltpu.async_copy(g1, out_slice(b), so1)
        return carry

    lax.fori_loop(0, _NK, round_body, 0)
    pltpu.make_async_copy(g0, out_slice(_NCHUNK - 2), so0).wait()
    pltpu.make_async_copy(g1, out_slice(_NCHUNK - 1), so1).wait()


def kernel(input_ids, values, gene_table, value_table):
    ids = input_ids.reshape(-1).astype(jnp.int32)
    vals = values.reshape(-1).astype(jnp.int32)
    vtab = jnp.pad(value_table, ((0, _VIN - _VALUE_VOCAB), (0, 0)))
    vtab = vtab[:, _PERM].astype(jnp.bfloat16).reshape(_VIN, _D // 2, 2)
    vtab = lax.bitcast_convert_type(vtab, jnp.int32)
    out = _sc_embed(ids, vals, gene_table, vtab)
    return out.reshape(_B, _L, _D)


# speedup vs baseline: 2.0290x; 1.5019x over previous
import functools

import jax
import jax.numpy as jnp
from jax import lax
from jax.experimental import pallas as pl
from jax.experimental.pallas import tpu as pltpu
from jax.experimental.pallas import tpu_sc as plsc

_GENE_VOCAB = 60697
_VALUE_VOCAB = 51
_VPAD = 56
_D = 512
_B, _L = 64, 1200
_N = _B * _L
_NC, _NS = 2, 16
_NW = _NC * _NS
_PER_W = _N // _NW
_C = 48
_NCHUNK = _PER_W // _C
_NK = _NCHUNK // 2

_mesh = plsc.VectorSubcoreMesh(core_axis_name="c", subcore_axis_name="s")


@functools.partial(
    pl.kernel,
    mesh=_mesh,
    out_type=jax.ShapeDtypeStruct((_N, _D), jnp.float32),
    scratch_types=[
        pltpu.VMEM((_PER_W,), jnp.int32),
        pltpu.VMEM((_PER_W,), jnp.int32),
        pltpu.VMEM((_C, _D), jnp.float32),
        pltpu.VMEM((_C, _D), jnp.float32),
        pltpu.VMEM((_C, _D), jnp.float32),
        pltpu.VMEM((_C, _D), jnp.float32),
        pltpu.HBM((_NW * _VPAD, _D), jnp.float32),
        pltpu.SemaphoreType.DMA,
        pltpu.SemaphoreType.DMA,
        pltpu.SemaphoreType.DMA,
        pltpu.SemaphoreType.DMA,
        pltpu.SemaphoreType.DMA,
        pltpu.SemaphoreType.DMA,
    ],
)
def _sc_embed(ids_hbm, vals_hbm, gene_hbm, vtab_hbm, out_hbm,
              gidx, vidx, g0, v0, g1, v1, vrep,
              sg0, sv0, sg1, sv1, so0, so1):
    wid = lax.axis_index("s") * _NC + lax.axis_index("c")
    base = wid * _PER_W

    pltpu.sync_copy(vtab_hbm.at[pl.ds(0, _C)], v0)
    pltpu.sync_copy(v0, vrep.at[pl.ds(wid * _VPAD, _C)])
    pltpu.sync_copy(vtab_hbm.at[pl.ds(_C, _VPAD - _C)], v0.at[pl.ds(0, _VPAD - _C)])
    pltpu.sync_copy(v0.at[pl.ds(0, _VPAD - _C)], vrep.at[pl.ds(wid * _VPAD + _C, _VPAD - _C)])

    pltpu.sync_copy(ids_hbm.at[pl.ds(base, _PER_W)], gidx)
    pltpu.sync_copy(vals_hbm.at[pl.ds(base, _PER_W)], vidx)

    vbase = wid * _VPAD

    def rebase(i, carry):
        sl = pl.ds(pl.multiple_of(i * 16, 16), 16)
        vidx[sl] = vidx[sl] + vbase
        return carry

    lax.fori_loop(0, _PER_W // 16, rebase, 0)

    def issue_gathers(ci, gbuf, vbuf, sg, sv):
        isl = pl.ds(pl.multiple_of(ci * _C, _C), _C)
        pltpu.async_copy(gene_hbm.at[gidx.at[isl]], gbuf, sg)
        pltpu.async_copy(vrep.at[vidx.at[isl]], vbuf, sv)

    def wait_gathers(ci, gbuf, vbuf, sg, sv):
        isl = pl.ds(pl.multiple_of(ci * _C, _C), _C)
        pltpu.make_async_copy(gene_hbm.at[gidx.at[isl]], gbuf, sg).wait()
        pltpu.make_async_copy(vrep.at[vidx.at[isl]], vbuf, sv).wait()

    def out_slice(ci):
        return out_hbm.at[pl.ds(pl.multiple_of(base + ci * _C, _C), _C)]

    def add_rows(gbuf, vbuf):
        def body(r, carry):
            for j in range(_D // 16):
                sl = pl.ds(j * 16, 16)
                gbuf[r, sl] = gbuf[r, sl] + vbuf[r, sl]
            return carry
        lax.fori_loop(0, _C, body, 0)

    issue_gathers(0, g0, v0, sg0, sv0)

    def round_body(k, carry):
        a = 2 * k
        b = a + 1

        @pl.when(k > 0)
        def _():
            pltpu.make_async_copy(g1, out_slice(b - 2), so1).wait()

        issue_gathers(b, g1, v1, sg1, sv1)

        wait_gathers(a, g0, v0, sg0, sv0)
        add_rows(g0, v0)
        pltpu.async_copy(g0, out_slice(a), so0)

        @pl.when(k < _NK - 1)
        def _():
            pltpu.make_async_copy(g0, out_slice(a), so0).wait()
            issue_gathers(a + 2, g0, v0, sg0, sv0)

        wait_gathers(b, g1, v1, sg1, sv1)
        add_rows(g1, v1)
        pltpu.async_copy(g1, out_slice(b), so1)
        return carry

    lax.fori_loop(0, _NK, round_body, 0)
    pltpu.make_async_copy(g0, out_slice(_NCHUNK - 2), so0).wait()
    pltpu.make_async_copy(g1, out_slice(_NCHUNK - 1), so1).wait()


def kernel(input_ids, values, gene_table, value_table):
    ids = input_ids.reshape(-1).astype(jnp.int32)
    vals = values.reshape(-1).astype(jnp.int32)
    vtab_padded = jnp.pad(value_table, ((0, _VPAD - _VALUE_VOCAB), (0, 0)))
    out = _sc_embed(ids, vals, gene_table, vtab_padded)
    return out.reshape(_B, _L, _D)
